# Initial kernel scaffold; baseline (speedup 1.0000x reference)
import jax
import jax.numpy as jnp
from jax.experimental import pallas as pl


def kernel(x, edge_index, batch, W11, b11, W12, b12, W21, b21, W22, b22, W31, b31, W32, b32, Wa, ba, Wb, bb, Wc, bc, Wr, br, Wcls, bcls, Wn, bn):
    raise NotImplementedError("write your pallas kernel here")


# speedup vs baseline: 9.1274x; 9.1274x over previous
import functools

import jax
import jax.numpy as jnp
from jax import lax
from jax.experimental import pallas as pl
from jax.experimental.pallas import tpu as pltpu
from jax.experimental.pallas import tpu_sc as plsc

_N = 10000
_E = 320000
_D = 128
_HD = _D // 2
_NC = 2
_NS = 16
_C = 100
_EPS = _E // _NS
_CH = _EPS // _C
_RPT = 632
_RPT_LAST = _N - 15 * _RPT
_RB = 1000


@functools.partial(
    pl.kernel,
    out_type=jax.ShapeDtypeStruct((_NC, _N, _HD), jnp.float32),
    mesh=plsc.VectorSubcoreMesh(core_axis_name="c", subcore_axis_name="s"),
    compiler_params=pltpu.CompilerParams(use_tc_tiling_on_sc=False),
    scratch_types=[
        pltpu.VMEM((_CH, _C), jnp.int32),
        pltpu.VMEM((_CH, _C), jnp.int32),
        pltpu.VMEM((_C, _HD), jnp.float32),
        pltpu.VMEM((_C, _HD), jnp.float32),
        pltpu.VMEM_SHARED((_N, _HD), jnp.float32),
        pltpu.SemaphoreType.DMA,
        pltpu.SemaphoreType.DMA,
    ],
)
def _sc_agg(x2_hbm, src_hbm, dst_hbm, z_hbm, out_hbm, src_v, dst_v, rows0,
            rows1, acc, sem0, sem1):
    c = lax.axis_index("c")
    s = lax.axis_index("s")
    @pl.when(s < _NS - 1)
    def _():
        pltpu.sync_copy(z_hbm.at[pl.ds(s * _RPT, _RPT)],
                        acc.at[pl.ds(s * _RPT, _RPT)])

    @pl.when(s == _NS - 1)
    def _():
        pltpu.sync_copy(z_hbm.at[pl.ds(15 * _RPT, _RPT_LAST)],
                        acc.at[pl.ds(15 * _RPT, _RPT_LAST)])

    pltpu.sync_copy(src_hbm.at[c * _NS + s], src_v)
    pltpu.sync_copy(dst_hbm.at[s], dst_v)
    plsc.subcore_barrier()

    pltpu.async_copy(x2_hbm.at[src_v.at[0]], rows0, sem0)

    def pair(i, carry):
        j = 2 * i
        pltpu.async_copy(x2_hbm.at[src_v.at[j + 1]], rows1, sem1)
        pltpu.make_async_copy(x2_hbm.at[src_v.at[j]], rows0, sem0).wait()
        pltpu.sync_copy(rows0, acc.at[dst_v.at[j]], add=True)

        @pl.when(j + 2 < _CH)
        def _():
            pltpu.async_copy(x2_hbm.at[src_v.at[j + 2]], rows0, sem0)

        pltpu.make_async_copy(x2_hbm.at[src_v.at[j + 1]], rows1, sem1).wait()
        pltpu.sync_copy(rows1, acc.at[dst_v.at[j + 1]], add=True)
        return carry

    lax.fori_loop(0, _CH // 2, pair, 0)

    plsc.subcore_barrier()

    @pl.when(s < _NS - 1)
    def _():
        pltpu.sync_copy(acc.at[pl.ds(s * _RPT, _RPT)],
                        out_hbm.at[c, pl.ds(s * _RPT, _RPT)])

    @pl.when(s == _NS - 1)
    def _():
        pltpu.sync_copy(acc.at[pl.ds(15 * _RPT, _RPT_LAST)],
                        out_hbm.at[c, pl.ds(15 * _RPT, _RPT_LAST)])


def _bdot(a, b):
    return jnp.dot(a.astype(jnp.bfloat16), b.astype(jnp.bfloat16),
                   preferred_element_type=jnp.float32)


def _mlp(x, agg, W1, b1, W2, b2):
    def body(x_ref, agg_ref, w1_ref, b1_ref, w2_ref, b2_ref, o_ref):
        h = x_ref[...] + jnp.concatenate([agg_ref[0], agg_ref[1]], axis=1)
        h1 = jnp.maximum(_bdot(h, w1_ref[...]) + b1_ref[...], 0.0)
        o_ref[...] = jnp.maximum(_bdot(h1, w2_ref[...]) + b2_ref[...], 0.0)

    return pl.pallas_call(
        body,
        grid=(_N // _RB,),
        in_specs=[
            pl.BlockSpec((_RB, _D), lambda i: (i, 0)),
            pl.BlockSpec((_NC, _RB, _HD), lambda i: (0, i, 0)),
            pl.BlockSpec((_D, _D), lambda i: (0, 0)),
            pl.BlockSpec((1, _D), lambda i: (0, 0)),
            pl.BlockSpec((_D, _D), lambda i: (0, 0)),
            pl.BlockSpec((1, _D), lambda i: (0, 0)),
        ],
        out_specs=pl.BlockSpec((_RB, _D), lambda i: (i, 0)),
        out_shape=jax.ShapeDtypeStruct((_N, _D), jnp.float32),
    )(x, agg, W1, b1.reshape(1, _D), W2, b2.reshape(1, _D))


def _attn(x3, Wa, ba, Wb, bb, Wc, bc, Wn, bn):
    def body(x_ref, wa, ba_, wb, bb_, wc, bc_, wn, bn_, a_ref, yn_ref):
        xb = x_ref[...]
        a = jnp.tanh(_bdot(xb, wa[...]) + ba_[...])
        b = jax.nn.sigmoid(_bdot(xb, wb[...]) + bb_[...])
        a_ref[...] = _bdot(a * b, wc[...]) + bc_[...]
        yn_ref[...] = _bdot(xb, wn[...]) + bn_[...]

    nn = bn.shape[0]
    return pl.pallas_call(
        body,
        grid=(_N // _RB,),
        in_specs=[
            pl.BlockSpec((_RB, _D), lambda i: (i, 0)),
            pl.BlockSpec((_D, _D), lambda i: (0, 0)),
            pl.BlockSpec((1, _D), lambda i: (0, 0)),
            pl.BlockSpec((_D, _D), lambda i: (0, 0)),
            pl.BlockSpec((1, _D), lambda i: (0, 0)),
            pl.BlockSpec((_D, 1), lambda i: (0, 0)),
            pl.BlockSpec((1, 1), lambda i: (0, 0)),
            pl.BlockSpec((_D, nn), lambda i: (0, 0)),
            pl.BlockSpec((1, nn), lambda i: (0, 0)),
        ],
        out_specs=[
            pl.BlockSpec((_RB, 1), lambda i: (i, 0)),
            pl.BlockSpec((_RB, nn), lambda i: (i, 0)),
        ],
        out_shape=[
            jax.ShapeDtypeStruct((_N, 1), jnp.float32),
            jax.ShapeDtypeStruct((_N, nn), jnp.float32),
        ],
    )(x3, Wa, ba.reshape(1, _D), Wb, bb.reshape(1, _D), Wc,
      bc.reshape(1, 1), Wn, bn.reshape(1, nn))


def _head(A_t, x3, Wr, br, Wcls, bcls):
    nc = bcls.shape[0]

    def body(a_ref, x_ref, wr, br_, wcls, bcls_, lg_ref, pr_ref, yh_ref,
             s_ref):
        A = a_ref[...]
        m = jnp.max(A, axis=1, keepdims=True)
        e = jnp.exp(A - m)
        p = e / jnp.sum(e, axis=1, keepdims=True)
        hp = _bdot(p, x_ref[...])
        hr = jnp.maximum(_bdot(hp, wr[...]) + br_[...], 0.0)
        lg = _bdot(hr, wcls[...]) + bcls_[...]
        lg_ref[...] = lg
        lm = jnp.max(lg, axis=1, keepdims=True)
        el = jnp.exp(lg - lm)
        pr_ref[...] = el / jnp.sum(el, axis=1, keepdims=True)
        col = lax.broadcasted_iota(jnp.int32, (1, nc), 1)
        yh_ref[...] = jnp.min(jnp.where(lg == lm, col, nc), axis=1,
                              keepdims=True)
        haz = jax.nn.sigmoid(lg)
        lgp = jnp.log(1.0 - haz)
        parts = [lgp[:, 0:1]]
        for k in range(1, nc):
            parts.append(parts[-1] + lgp[:, k:k + 1])
        s_ref[...] = jnp.exp(jnp.concatenate(parts, axis=1))

    return pl.pallas_call(
        body,
        out_shape=[
            jax.ShapeDtypeStruct((1, nc), jnp.float32),
            jax.ShapeDtypeStruct((1, nc), jnp.float32),
            jax.ShapeDtypeStruct((1, 1), jnp.int32),
            jax.ShapeDtypeStruct((1, nc), jnp.float32),
        ],
    )(A_t, x3, Wr, br.reshape(1, _D), Wcls, bcls.reshape(1, nc))


def kernel(x, edge_index, batch, W11, b11, W12, b12, W21, b21, W22, b22,
           W31, b31, W32, b32, Wa, ba, Wb, bb, Wc, bc, Wr, br, Wcls, bcls,
           Wn, bn):
    src = edge_index[0]
    srca = jnp.stack([src * 2, src * 2 + 1]).reshape(_NC * _NS, _CH, _C)
    dst = edge_index[1].reshape(_NS, _CH, _C)
    z = jnp.zeros((_N, _HD), jnp.float32)

    def agg(h):
        return _sc_agg(h.reshape(2 * _N, _HD), srca, dst, z)

    x1 = _mlp(x, agg(x), W11, b11, W12, b12)
    x2 = _mlp(x1, agg(x1), W21, b21, W22, b22)
    x3 = _mlp(x2, agg(x2), W31, b31, W32, b32)
    A, Y_node = _attn(x3, Wa, ba, Wb, bb, Wc, bc, Wn, bn)
    A_raw = A.reshape(1, _N)
    logits, Y_prob, Y_hat, S = _head(A_raw, x3, Wr, br, Wcls, bcls)
    return (logits, Y_prob, Y_hat, A_raw, S, Y_node)
